# Initial kernel scaffold; baseline (speedup 1.0000x reference)
import jax
import jax.numpy as jnp
from jax.experimental import pallas as pl


def kernel(x, W, b):
    raise NotImplementedError("write your pallas kernel here")


# speedup vs baseline: 1.8109x; 1.8109x over previous
import functools

import jax
import jax.numpy as jnp
from jax.experimental import pallas as pl

BT = 512


def _router_kernel(x_ref, w_ref, b_ref, gate_ref, idx_ref):
    logits = jnp.dot(x_ref[...], w_ref[...].T,
                     preferred_element_type=jnp.float32) + b_ref[...]
    m = jnp.max(logits, axis=1)
    idx = jnp.argmax(logits, axis=1).astype(jnp.int32)
    s = jnp.sum(jnp.exp(logits - m[:, None]), axis=1)
    gate_ref[...] = 1.0 / s
    idx_ref[...] = idx


@jax.jit
def kernel(x, W, b):
    tokens, hidden = x.shape
    num_experts = W.shape[0]
    b2 = b.reshape(1, num_experts)
    grid = (tokens // BT,)
    gate, idx = pl.pallas_call(
        _router_kernel,
        grid=grid,
        in_specs=[
            pl.BlockSpec((BT, hidden), lambda i: (i, 0)),
            pl.BlockSpec((num_experts, hidden), lambda i: (0, 0)),
            pl.BlockSpec((1, num_experts), lambda i: (0, 0)),
        ],
        out_specs=[
            pl.BlockSpec((BT,), lambda i: (i,)),
            pl.BlockSpec((BT,), lambda i: (i,)),
        ],
        out_shape=[
            jax.ShapeDtypeStruct((tokens,), jnp.float32),
            jax.ShapeDtypeStruct((tokens,), jnp.int32),
        ],
    )(x, W, b2)
    return gate.reshape(tokens, 1), idx.reshape(tokens, 1).astype(jnp.int64)
